# baseline (speedup 1.0000x reference)
import functools

import jax
import jax.numpy as jnp
from jax import lax
from jax.experimental import pallas as pl
from jax.experimental.pallas import tpu as pltpu
from jax.experimental.pallas import tpu_sc as plsc

_NC = 2
_NS = 16


def _select_kernel(xyz1t_ref, xyz2_ref, p1t_ref, w1t_ref, b1r_ref,
                   i0_ref, i1_ref, i2_ref, w0_ref, w1_ref, w2_ref, p1_ref,
                   *, M, TN):
    b = pl.program_id(0)

    @pl.when(pl.program_id(1) == 0)
    def _():
        p1_ref[0] = (
            jnp.dot(p1t_ref[0], w1t_ref[...],
                    preferred_element_type=jnp.float32)
            + b1r_ref[...]
        )

    x1t = xyz1t_ref[0]
    x2 = xyz2_ref[0]

    d0 = x1t[:, 0][:, None] - x2[0][None, :]
    d1 = x1t[:, 1][:, None] - x2[1][None, :]
    d2 = x1t[:, 2][:, None] - x2[2][None, :]
    D = d0 * d0 + d1 * d1 + d2 * d2

    m0 = jnp.min(D, axis=0, keepdims=True)
    D1 = jnp.where(D == m0, jnp.inf, D)
    m1 = jnp.min(D1, axis=0, keepdims=True)
    D2 = jnp.where(D1 == m1, jnp.inf, D1)
    m2 = jnp.min(D2, axis=0, keepdims=True)

    r0 = 1.0 / (m0 + 0.1)
    r1 = 1.0 / (m1 + 0.1)
    r2 = 1.0 / (m2 + 0.1)
    norm = r0 + r1 + r2
    w0_ref[0, 0] = r0 / norm
    w1_ref[0, 0] = r1 / norm
    w2_ref[0, 0] = r2 / norm

    iota0 = lax.broadcasted_iota(jnp.int32, (M, TN), 0)
    gbase = b * M
    for m_val, i_ref in ((m0, i0_ref), (m1, i1_ref), (m2, i2_ref)):
        idx = jnp.max(jnp.where(D == m_val, iota0, -1), axis=0, keepdims=True)
        i_ref[0, 0] = jnp.clip(idx, 0, M - 1) + gbase


def _combine_kernel(g0_ref, g1_ref, g2_ref, w0_ref, w1_ref, w2_ref,
                    p2_ref, w2m_ref, b2c_ref, out_ref):
    w0 = jnp.transpose(w0_ref[0, 0])
    w1 = jnp.transpose(w1_ref[0, 0])
    w2 = jnp.transpose(w2_ref[0, 0])
    ws = g0_ref[0, 0] * w0 + g1_ref[0, 0] * w1 + g2_ref[0, 0] * w2
    p2 = (
        jnp.dot(w2m_ref[...], p2_ref[0], preferred_element_type=jnp.float32)
        + b2c_ref[...]
    )
    out_ref[0] = jnp.transpose(ws) + p2


def _make_sc_gather(ROWS, O, CH, n_workers):
    rpw = ROWS // n_workers
    nchunk = rpw // CH
    mesh = plsc.VectorSubcoreMesh(core_axis_name="c", subcore_axis_name="s")
    f32 = jnp.float32

    @functools.partial(
        pl.kernel, mesh=mesh,
        out_type=(jax.ShapeDtypeStruct((ROWS, O), f32),) * 3,
        scratch_types=(
            [pltpu.VMEM((CH,), jnp.int32)] * 3
            + [pltpu.VMEM((CH, O), f32)] * 3
            + [pltpu.SemaphoreType.DMA]
        ),
    )
    def sc_gather(i0h, i1h, i2h, p1h, g0h, g1h, g2h,
                  iv0, iv1, iv2, gv0, gv1, gv2, sem):
        wid = lax.axis_index("s") * _NC + lax.axis_index("c")
        for c in range(nchunk):
            base = wid * rpw + c * CH
            sl = pl.ds(base, CH)
            pltpu.sync_copy(i0h.at[sl], iv0)
            pltpu.sync_copy(i1h.at[sl], iv1)
            pltpu.sync_copy(i2h.at[sl], iv2)
            c0 = pltpu.async_copy(p1h.at[iv0], gv0, sem)
            c1 = pltpu.async_copy(p1h.at[iv1], gv1, sem)
            c2 = pltpu.async_copy(p1h.at[iv2], gv2, sem)
            c0.wait()
            c1.wait()
            c2.wait()
            pltpu.sync_copy(gv0, g0h.at[sl])
            pltpu.sync_copy(gv1, g1h.at[sl])
            pltpu.sync_copy(gv2, g2h.at[sl])

    return sc_gather


def _chunk(h, Bc, xyz_1t, xyz_2, points_1t, points_2, w1t, b1r, W2, b2c,
           M, N, C, O, TN):
    NB = N // TN
    ROWS = Bc * N
    b0 = h * Bc

    idx_w_specs = pl.BlockSpec((1, 1, 1, TN), lambda b, nb: (b, nb, 0, 0))
    sel_out = pl.pallas_call(
        functools.partial(_select_kernel, M=M, TN=TN),
        grid=(Bc, NB),
        in_specs=[
            pl.BlockSpec((1, M, 3), lambda b, nb: (b0 + b, 0, 0)),
            pl.BlockSpec((1, 3, TN), lambda b, nb: (b0 + b, 0, nb)),
            pl.BlockSpec((1, M, C), lambda b, nb: (b0 + b, 0, 0)),
            pl.BlockSpec((C, O), lambda b, nb: (0, 0)),
            pl.BlockSpec((1, O), lambda b, nb: (0, 0)),
        ],
        out_specs=(
            [idx_w_specs] * 6
            + [pl.BlockSpec((1, M, O), lambda b, nb: (b, 0, 0))]
        ),
        out_shape=(
            [jax.ShapeDtypeStruct((Bc, NB, 1, TN), jnp.int32)] * 3
            + [jax.ShapeDtypeStruct((Bc, NB, 1, TN), jnp.float32)] * 3
            + [jax.ShapeDtypeStruct((Bc, M, O), jnp.float32)]
        ),
    )(xyz_1t, xyz_2, points_1t, w1t, b1r)
    i0, i1, i2, w0, w1, w2, p1rows = sel_out

    sc_gather = _make_sc_gather(ROWS, O, CH=128, n_workers=_NC * _NS)
    g0, g1, g2 = sc_gather(
        i0.reshape(ROWS), i1.reshape(ROWS), i2.reshape(ROWS),
        p1rows.reshape(Bc * M, O))
    TNC = 1024
    NBC = N // TNC
    g0 = g0.reshape(Bc, NBC, TNC, O)
    g1 = g1.reshape(Bc, NBC, TNC, O)
    g2 = g2.reshape(Bc, NBC, TNC, O)
    w0 = w0.reshape(Bc, NBC, 1, TNC)
    w1 = w1.reshape(Bc, NBC, 1, TNC)
    w2 = w2.reshape(Bc, NBC, 1, TNC)

    g_spec = pl.BlockSpec((1, 1, TNC, O), lambda b, nb: (b, nb, 0, 0))
    wc_spec = pl.BlockSpec((1, 1, 1, TNC), lambda b, nb: (b, nb, 0, 0))
    return pl.pallas_call(
        _combine_kernel,
        grid=(Bc, NBC),
        in_specs=[
            g_spec, g_spec, g_spec,
            wc_spec, wc_spec, wc_spec,
            pl.BlockSpec((1, O, TNC), lambda b, nb: (b0 + b, 0, nb)),
            pl.BlockSpec((O, O), lambda b, nb: (0, 0)),
            pl.BlockSpec((O, 1), lambda b, nb: (0, 0)),
        ],
        out_specs=pl.BlockSpec((1, O, TNC), lambda b, nb: (b, 0, nb)),
        out_shape=jax.ShapeDtypeStruct((Bc, O, N), jnp.float32),
    )(g0, g1, g2, w0, w1, w2, points_2, W2, b2c)


def kernel(xyz_1, xyz_2, points_1, points_2, W1, b1, W2, b2):
    B, _, M = xyz_1.shape
    N = xyz_2.shape[2]
    C = points_1.shape[1]
    O = W1.shape[0]
    TN = 1024
    CHUNKS = 4
    Bc = B // CHUNKS

    xyz_1t = jnp.transpose(xyz_1, (0, 2, 1))
    points_1t = jnp.transpose(points_1, (0, 2, 1))
    w1t = W1.T
    b1r = b1.reshape(1, O)
    b2c = b2.reshape(O, 1)

    outs = [
        _chunk(h, Bc, xyz_1t, xyz_2, points_1t, points_2,
               w1t, b1r, W2, b2c, M, N, C, O, TN)
        for h in range(CHUNKS)
    ]
    out = jnp.concatenate(outs, axis=0) if CHUNKS > 1 else outs[0]

    return (xyz_2, out)


# speedup vs baseline: 1.0065x; 1.0065x over previous
import functools

import jax
import jax.numpy as jnp
from jax import lax
from jax.experimental import pallas as pl
from jax.experimental.pallas import tpu as pltpu
from jax.experimental.pallas import tpu_sc as plsc

_NC = 2
_NS = 16


def _select_kernel(xyz1t_ref, xyz2_ref, p1t_ref, w1t_ref, b1r_ref,
                   i0_ref, i1_ref, i2_ref, w0_ref, w1_ref, w2_ref, p1_ref,
                   *, M, TN):
    b = pl.program_id(0)

    @pl.when(pl.program_id(1) == 0)
    def _():
        p1_ref[0] = (
            jnp.dot(p1t_ref[0], w1t_ref[...],
                    preferred_element_type=jnp.float32)
            + b1r_ref[...]
        )

    x1t = xyz1t_ref[0]
    x2 = xyz2_ref[0]

    d0 = x1t[:, 0][:, None] - x2[0][None, :]
    d1 = x1t[:, 1][:, None] - x2[1][None, :]
    d2 = x1t[:, 2][:, None] - x2[2][None, :]
    D = d0 * d0 + d1 * d1 + d2 * d2

    m0 = jnp.min(D, axis=0, keepdims=True)
    D1 = jnp.where(D == m0, jnp.inf, D)
    m1 = jnp.min(D1, axis=0, keepdims=True)
    D2 = jnp.where(D1 == m1, jnp.inf, D1)
    m2 = jnp.min(D2, axis=0, keepdims=True)

    r0 = 1.0 / (m0 + 0.1)
    r1 = 1.0 / (m1 + 0.1)
    r2 = 1.0 / (m2 + 0.1)
    norm = r0 + r1 + r2
    w0_ref[0, 0] = r0 / norm
    w1_ref[0, 0] = r1 / norm
    w2_ref[0, 0] = r2 / norm

    iota0 = lax.broadcasted_iota(jnp.int32, (M, TN), 0)
    gbase = b * M
    for m_val, i_ref in ((m0, i0_ref), (m1, i1_ref), (m2, i2_ref)):
        idx = jnp.max(jnp.where(D == m_val, iota0, -1), axis=0, keepdims=True)
        i_ref[0, 0] = jnp.clip(idx, 0, M - 1) + gbase


def _combine_kernel(g0_ref, g1_ref, g2_ref, w0_ref, w1_ref, w2_ref,
                    p2_ref, w2m_ref, b2c_ref, out_ref):
    w0 = jnp.transpose(w0_ref[0, 0])
    w1 = jnp.transpose(w1_ref[0, 0])
    w2 = jnp.transpose(w2_ref[0, 0])
    ws = g0_ref[0, 0] * w0 + g1_ref[0, 0] * w1 + g2_ref[0, 0] * w2
    p2 = (
        jnp.dot(w2m_ref[...], p2_ref[0], preferred_element_type=jnp.float32)
        + b2c_ref[...]
    )
    out_ref[0] = jnp.transpose(ws) + p2


def _make_sc_gather(ROWS, O, CH, n_workers):
    rpw = ROWS // n_workers
    nchunk = rpw // CH
    mesh = plsc.VectorSubcoreMesh(core_axis_name="c", subcore_axis_name="s")
    f32 = jnp.float32

    @functools.partial(
        pl.kernel, mesh=mesh,
        out_type=(jax.ShapeDtypeStruct((ROWS, O), f32),) * 3,
        scratch_types=(
            [pltpu.VMEM((CH,), jnp.int32)] * 3
            + [pltpu.VMEM((CH, O), f32)] * 3
            + [pltpu.SemaphoreType.DMA]
        ),
    )
    def sc_gather(i0h, i1h, i2h, p1h, g0h, g1h, g2h,
                  iv0, iv1, iv2, gv0, gv1, gv2, sem):
        wid = lax.axis_index("s") * _NC + lax.axis_index("c")
        for c in range(nchunk):
            base = wid * rpw + c * CH
            sl = pl.ds(base, CH)
            pltpu.sync_copy(i0h.at[sl], iv0)
            pltpu.sync_copy(i1h.at[sl], iv1)
            pltpu.sync_copy(i2h.at[sl], iv2)
            c0 = pltpu.async_copy(p1h.at[iv0], gv0, sem)
            c1 = pltpu.async_copy(p1h.at[iv1], gv1, sem)
            c2 = pltpu.async_copy(p1h.at[iv2], gv2, sem)
            c0.wait()
            c1.wait()
            c2.wait()
            pltpu.sync_copy(gv0, g0h.at[sl])
            pltpu.sync_copy(gv1, g1h.at[sl])
            pltpu.sync_copy(gv2, g2h.at[sl])

    return sc_gather


def _chunk(h, Bc, xyz_1t, xyz_2, points_1t, points_2, w1t, b1r, W2, b2c,
           M, N, C, O, TN):
    NB = N // TN
    ROWS = Bc * N
    b0 = h * Bc

    idx_w_specs = pl.BlockSpec((1, 1, 1, TN), lambda b, nb: (b, nb, 0, 0))
    sel_out = pl.pallas_call(
        functools.partial(_select_kernel, M=M, TN=TN),
        grid=(Bc, NB),
        in_specs=[
            pl.BlockSpec((1, M, 3), lambda b, nb: (b0 + b, 0, 0)),
            pl.BlockSpec((1, 3, TN), lambda b, nb: (b0 + b, 0, nb)),
            pl.BlockSpec((1, M, C), lambda b, nb: (b0 + b, 0, 0)),
            pl.BlockSpec((C, O), lambda b, nb: (0, 0)),
            pl.BlockSpec((1, O), lambda b, nb: (0, 0)),
        ],
        out_specs=(
            [idx_w_specs] * 6
            + [pl.BlockSpec((1, M, O), lambda b, nb: (b, 0, 0))]
        ),
        out_shape=(
            [jax.ShapeDtypeStruct((Bc, NB, 1, TN), jnp.int32)] * 3
            + [jax.ShapeDtypeStruct((Bc, NB, 1, TN), jnp.float32)] * 3
            + [jax.ShapeDtypeStruct((Bc, M, O), jnp.float32)]
        ),
    )(xyz_1t, xyz_2, points_1t, w1t, b1r)
    i0, i1, i2, w0, w1, w2, p1rows = sel_out

    sc_gather = _make_sc_gather(ROWS, O, CH=128, n_workers=_NC * _NS)
    g0, g1, g2 = sc_gather(
        i0.reshape(ROWS), i1.reshape(ROWS), i2.reshape(ROWS),
        p1rows.reshape(Bc * M, O))
    TNC = 1024
    NBC = N // TNC
    g0 = g0.reshape(Bc, NBC, TNC, O)
    g1 = g1.reshape(Bc, NBC, TNC, O)
    g2 = g2.reshape(Bc, NBC, TNC, O)
    w0 = w0.reshape(Bc, NBC, 1, TNC)
    w1 = w1.reshape(Bc, NBC, 1, TNC)
    w2 = w2.reshape(Bc, NBC, 1, TNC)

    g_spec = pl.BlockSpec((1, 1, TNC, O), lambda b, nb: (b, nb, 0, 0))
    wc_spec = pl.BlockSpec((1, 1, 1, TNC), lambda b, nb: (b, nb, 0, 0))
    return pl.pallas_call(
        _combine_kernel,
        grid=(Bc, NBC),
        in_specs=[
            g_spec, g_spec, g_spec,
            wc_spec, wc_spec, wc_spec,
            pl.BlockSpec((1, O, TNC), lambda b, nb: (b0 + b, 0, nb)),
            pl.BlockSpec((O, O), lambda b, nb: (0, 0)),
            pl.BlockSpec((O, 1), lambda b, nb: (0, 0)),
        ],
        out_specs=pl.BlockSpec((1, O, TNC), lambda b, nb: (b, 0, nb)),
        out_shape=jax.ShapeDtypeStruct((Bc, O, N), jnp.float32),
    )(g0, g1, g2, w0, w1, w2, points_2, W2, b2c)


def kernel(xyz_1, xyz_2, points_1, points_2, W1, b1, W2, b2):
    B, _, M = xyz_1.shape
    N = xyz_2.shape[2]
    C = points_1.shape[1]
    O = W1.shape[0]
    TN = 512
    CHUNKS = 4
    Bc = B // CHUNKS

    xyz_1t = jnp.transpose(xyz_1, (0, 2, 1))
    points_1t = jnp.transpose(points_1, (0, 2, 1))
    w1t = W1.T
    b1r = b1.reshape(1, O)
    b2c = b2.reshape(O, 1)

    outs = [
        _chunk(h, Bc, xyz_1t, xyz_2, points_1t, points_2,
               w1t, b1r, W2, b2c, M, N, C, O, TN)
        for h in range(CHUNKS)
    ]
    out = jnp.concatenate(outs, axis=0) if CHUNKS > 1 else outs[0]

    return (xyz_2, out)
